# baseline (speedup 1.0000x reference)
import functools

import jax
import jax.numpy as jnp
from jax import lax
from jax.experimental import pallas as pl
from jax.experimental.pallas import tpu as pltpu
from jax.experimental.pallas import tpu_sc as plsc

_N = 100000
_E = 6400000
_F = 8
_CH = 200
_NW = 32
_EPW = _E // _NW
_CHUNKS = _EPW // _CH
_RPT = _N // 16
_RWIN = 6256
_PR = _N // 16
_BR = 1250

_mesh = plsc.VectorSubcoreMesh(core_axis_name="c", subcore_axis_name="s")
_sc_params = pltpu.CompilerParams(use_tc_tiling_on_sc=False)


def _row_window(s):
    r0 = s * _RPT
    r0a = (r0 // 8) * 8
    return r0a


_NS = 8
_G = 10
_GB = _G * _CH
_BLK = 2 * _G
_NIT = _CHUNKS // _BLK


@functools.partial(
    pl.kernel,
    out_type=jax.ShapeDtypeStruct((2, _N), jnp.float32),
    mesh=_mesh,
    compiler_params=_sc_params,
    scratch_types=(
        [pltpu.VMEM((_GB,), jnp.int32) for _ in range(2)]
        + [pltpu.VMEM((_CH,), jnp.float32)]
        + [pltpu.SemaphoreType.DMA for _ in range(_NS + 2)]
        + [pltpu.VMEM_SHARED((_N,), jnp.float32)]
    ),
)
def _sc_deg(ei_hbm, ones_hbm, zeros_hbm, out_hbm, *refs):
    idx_ab = refs[0:2]
    ones_rows = refs[2]
    sem_s = refs[3:3 + _NS]
    sem_i = refs[3 + _NS:5 + _NS]
    deg_sh = refs[5 + _NS]
    c = lax.axis_index("c")
    s = lax.axis_index("s")
    wid = c * 16 + s
    r0a = _row_window(s)
    pltpu.sync_copy(ones_hbm, ones_rows)
    pltpu.sync_copy(zeros_hbm, deg_sh.at[pl.ds(r0a, _RWIN)])
    plsc.subcore_barrier()
    e_base = wid * _EPW

    def _load_grp(p, off):
        pltpu.async_copy(ei_hbm.at[1, pl.ds(off, _GB)], idx_ab[p], sem_i[p])

    def _wait_grp(p, off):
        pltpu.make_async_copy(ei_hbm.at[1, pl.ds(off, _GB)], idx_ab[p],
                              sem_i[p]).wait()

    def _slice_d(t):
        return idx_ab[t // _G].at[pl.ds((t % _G) * _CH, _CH)]

    def _scat(t):
        pltpu.async_copy(ones_rows, deg_sh.at[_slice_d(t)], sem_s[t % _NS],
                         add=True)

    def _wait_scat(t):
        pltpu.make_async_copy(ones_rows, deg_sh.at[_slice_d(t)],
                              sem_s[t % _NS]).wait()

    def _block(base, it, first):
        for t in range(_BLK):
            if not first or t >= _NS:
                _wait_scat((t - _NS) % _BLK)
            if t == _NS:
                _load_grp(1, base + _G * _CH)
            if t == _G:
                _wait_grp(1, base + _G * _CH)
            if t == _G + _NS:
                _load_grp(0, e_base + ((it + 1) % _NIT) * _BLK * _CH)
            if t == 0:
                _wait_grp(0, base)
            _scat(t)

    _load_grp(0, e_base)
    _block(e_base, 0, True)

    def body(it, carry):
        _block(e_base + it * _BLK * _CH, it, False)
        return carry

    lax.fori_loop(1, _NIT, body, 0)
    _wait_grp(0, e_base)
    for t in range(_BLK - _NS, _BLK):
        _wait_scat(t)
    plsc.subcore_barrier()
    pltpu.sync_copy(deg_sh.at[pl.ds(r0a, _RWIN)],
                    out_hbm.at[c, pl.ds(r0a, _RWIN)])


@functools.partial(
    pl.kernel,
    out_type=jax.ShapeDtypeStruct((2, _N, _F), jnp.float32),
    mesh=_mesh,
    compiler_params=_sc_params,
    scratch_types=(
        [pltpu.VMEM((_GB,), jnp.int32) for _ in range(4)]
        + [pltpu.VMEM((_CH, _F), jnp.float32) for _ in range(_NS)]
        + [pltpu.SemaphoreType.DMA for _ in range(2 * _NS + 2)]
        + [pltpu.VMEM_SHARED((_N, _F), jnp.float32)]
        + [pltpu.VMEM_SHARED((_N, _F), jnp.float32)]
    ),
)
def _sc_agg(y_hbm, ei_hbm, zeros_hbm, out_hbm, *refs):
    idx_s_ab = refs[0:2]
    idx_d_ab = refs[2:4]
    rows = refs[4:4 + _NS]
    sem_g = refs[4 + _NS:4 + 2 * _NS]
    sem_s = refs[4 + 2 * _NS:4 + 3 * _NS]
    sem_i = refs[4 + 3 * _NS:6 + 3 * _NS]
    y_sh = refs[6 + 3 * _NS]
    z_sh = refs[7 + 3 * _NS]
    c = lax.axis_index("c")
    s = lax.axis_index("s")
    wid = c * 16 + s
    r0a = _row_window(s)
    pltpu.sync_copy(y_hbm.at[pl.ds(r0a, _RWIN)], y_sh.at[pl.ds(r0a, _RWIN)])
    pltpu.sync_copy(zeros_hbm, z_sh.at[pl.ds(r0a, _RWIN)])
    plsc.subcore_barrier()
    e_base = wid * _EPW

    def _load_grp(p, off):
        pltpu.async_copy(ei_hbm.at[0, pl.ds(off, _GB)], idx_s_ab[p], sem_i[p])
        pltpu.async_copy(ei_hbm.at[1, pl.ds(off, _GB)], idx_d_ab[p], sem_i[p])

    def _wait_grp(p, off):
        pltpu.make_async_copy(ei_hbm.at[0, pl.ds(off, _GB)], idx_s_ab[p],
                              sem_i[p]).wait()
        pltpu.make_async_copy(ei_hbm.at[1, pl.ds(off, _GB)], idx_d_ab[p],
                              sem_i[p]).wait()

    def _slice_s(t):
        return idx_s_ab[t // _G].at[pl.ds((t % _G) * _CH, _CH)]

    def _slice_d(t):
        return idx_d_ab[t // _G].at[pl.ds((t % _G) * _CH, _CH)]

    def _gather(t):
        pltpu.async_copy(y_sh.at[_slice_s(t)], rows[t % _NS], sem_g[t % _NS])

    def _wait_gather(t):
        pltpu.make_async_copy(y_sh.at[_slice_s(t)], rows[t % _NS],
                              sem_g[t % _NS]).wait()

    def _scatter(t):
        pltpu.async_copy(rows[t % _NS], z_sh.at[_slice_d(t)], sem_s[t % _NS],
                         add=True)

    def _wait_scatter(t):
        pltpu.make_async_copy(rows[t % _NS], z_sh.at[_slice_d(t)],
                              sem_s[t % _NS]).wait()

    def _block(base, it, first):
        for t in range(_BLK):
            if not first or t >= _NS:
                _wait_scatter((t - _NS) % _BLK)
            if t == _NS:
                _load_grp(1, base + _G * _CH)
            if t == _G:
                _wait_grp(1, base + _G * _CH)
            if t == _G + _NS:
                _load_grp(0, e_base + ((it + 1) % _NIT) * _BLK * _CH)
            if t == 0:
                _wait_grp(0, base)
            if not first or t >= 2:
                _wait_gather((t - 2) % _BLK)
                _scatter((t - 2) % _BLK)
            _gather(t)

    _load_grp(0, e_base)
    _block(e_base, 0, True)

    def body(it, carry):
        _block(e_base + it * _BLK * _CH, it, False)
        return carry

    lax.fori_loop(1, _NIT, body, 0)
    _wait_grp(0, e_base)
    for t in (_BLK - 2, _BLK - 1):
        _wait_gather(t)
        _scatter(t)
    for t in range(_BLK - _NS, _BLK):
        _wait_scatter(t)
    plsc.subcore_barrier()
    pltpu.sync_copy(z_sh.at[pl.ds(r0a, _RWIN)],
                    out_hbm.at[c, pl.ds(r0a, _RWIN)])


def _t1_body(degp_ref, xpad_ref, sbc_ref, dinv_ref, xd_ref):
    deg = degp_ref[0] + degp_ref[1] + 1.0
    dinv16 = lax.rsqrt(deg)
    dinv = jnp.dot(dinv16, sbc_ref[...], preferred_element_type=jnp.float32)
    dinv_ref[...] = dinv
    xd_ref[...] = xpad_ref[...] * dinv


_t1 = pl.pallas_call(
    _t1_body,
    out_shape=[
        jax.ShapeDtypeStruct((_PR, 128), jnp.float32),
        jax.ShapeDtypeStruct((_PR, 128), jnp.float32),
    ],
)


def _t2_body(zp_ref, xd_ref, dinv_ref, w1_ref, b1_ref, w2_ref, y2_ref):
    w = (zp_ref[0] + zp_ref[1] + xd_ref[...]) * dinv_ref[...]
    h1 = jnp.dot(w, w1_ref[...], preferred_element_type=jnp.float32)
    h1 = jnp.maximum(h1 + b1_ref[...], 0.0)
    y2 = jnp.dot(h1, w2_ref[...], preferred_element_type=jnp.float32)
    y2_ref[...] = y2 * dinv_ref[...]


_t2 = pl.pallas_call(
    _t2_body,
    out_shape=jax.ShapeDtypeStruct((_PR, 128), jnp.float32),
)


def _t3_body(zp_ref, y2_ref, dinv_ref, b2_ref, wfc_ref, bfc_ref, out_ref):
    h2 = (zp_ref[0] + zp_ref[1] + y2_ref[...]) * dinv_ref[...]
    h2 = jnp.maximum(h2 + b2_ref[...], 0.0)
    o = jnp.dot(h2, wfc_ref[...], preferred_element_type=jnp.float32)
    out_ref[...] = jax.nn.sigmoid(o + bfc_ref[...])


_t3 = pl.pallas_call(
    _t3_body,
    out_shape=jax.ShapeDtypeStruct((_PR, 16), jnp.float32),
)


def kernel(x, edge_index, W1, b1, W2, b2, Wfc, bfc):
    f32 = jnp.float32
    ei = edge_index.astype(jnp.int32)
    zeros8 = jnp.zeros((_RWIN, _F), f32)
    ones1 = jnp.ones((_CH,), f32)
    zeros1 = jnp.zeros((_RWIN,), f32)

    degp = _sc_deg(ei, ones1, zeros1)
    xpad = jnp.concatenate([x.astype(f32), jnp.zeros((_N, 2), f32)], axis=1)
    sbc = jnp.kron(jnp.eye(16, dtype=f32), jnp.ones((1, 8), f32))
    dinv_p, xd_p = _t1(degp.reshape(2, _PR, 16), xpad.reshape(_PR, 128), sbc)

    z1p = _sc_agg(xd_p.reshape(_N, _F), ei, zeros8)

    eye16 = jnp.eye(16, dtype=f32)
    w1pad = jnp.concatenate([W1.astype(f32), jnp.zeros((2, 16), f32)], axis=0)
    w1big = jnp.kron(eye16, w1pad)
    b1big = jnp.tile(b1.astype(f32), 16).reshape(1, 256)
    w2big = jnp.kron(eye16, W2.astype(f32))
    y2_p = _t2(z1p.reshape(2, _PR, 128), xd_p, dinv_p, w1big, b1big, w2big)

    z2p = _sc_agg(y2_p.reshape(_N, _F), ei, zeros8)

    b2big = jnp.tile(b2.astype(f32), 16).reshape(1, 128)
    wfcbig = jnp.kron(eye16, Wfc.astype(f32))
    bfcbig = jnp.tile(bfc.astype(f32), 16).reshape(1, 16)
    out_p = _t3(z2p.reshape(2, _PR, 128), y2_p, dinv_p, b2big, wfcbig, bfcbig)
    return out_p.reshape(_N, 1)


# speedup vs baseline: 1.1314x; 1.1314x over previous
import functools

import jax
import jax.numpy as jnp
from jax import lax
from jax.experimental import pallas as pl
from jax.experimental.pallas import tpu as pltpu
from jax.experimental.pallas import tpu_sc as plsc

_N = 100000
_E = 6400000
_F = 8
_CH = 400
_NW = 32
_EPW = _E // _NW
_CHUNKS = _EPW // _CH
_RPT = _N // 16
_RWIN = 6256
_PR = _N // 16
_BR = 1250

_mesh = plsc.VectorSubcoreMesh(core_axis_name="c", subcore_axis_name="s")
_sc_params = pltpu.CompilerParams(use_tc_tiling_on_sc=False)


def _row_window(s):
    r0 = s * _RPT
    r0a = (r0 // 8) * 8
    return r0a


_NS = 4
_G = 10
_GB = _G * _CH
_BLK = 2 * _G
_NIT = _CHUNKS // _BLK


@functools.partial(
    pl.kernel,
    out_type=jax.ShapeDtypeStruct((2, _N), jnp.float32),
    mesh=_mesh,
    compiler_params=_sc_params,
    scratch_types=(
        [pltpu.VMEM((_GB,), jnp.int32) for _ in range(2)]
        + [pltpu.VMEM((_CH,), jnp.float32)]
        + [pltpu.SemaphoreType.DMA for _ in range(_NS + 2)]
        + [pltpu.VMEM_SHARED((_N,), jnp.float32)]
    ),
)
def _sc_deg(ei_hbm, ones_hbm, zeros_hbm, out_hbm, *refs):
    idx_ab = refs[0:2]
    ones_rows = refs[2]
    sem_s = refs[3:3 + _NS]
    sem_i = refs[3 + _NS:5 + _NS]
    deg_sh = refs[5 + _NS]
    c = lax.axis_index("c")
    s = lax.axis_index("s")
    wid = c * 16 + s
    r0a = _row_window(s)
    pltpu.sync_copy(ones_hbm, ones_rows)
    pltpu.sync_copy(zeros_hbm, deg_sh.at[pl.ds(r0a, _RWIN)])
    plsc.subcore_barrier()
    e_base = wid * _EPW

    def _load_grp(p, off):
        pltpu.async_copy(ei_hbm.at[1, pl.ds(off, _GB)], idx_ab[p], sem_i[p])

    def _wait_grp(p, off):
        pltpu.make_async_copy(ei_hbm.at[1, pl.ds(off, _GB)], idx_ab[p],
                              sem_i[p]).wait()

    def _slice_d(t):
        return idx_ab[t // _G].at[pl.ds((t % _G) * _CH, _CH)]

    def _scat(t):
        pltpu.async_copy(ones_rows, deg_sh.at[_slice_d(t)], sem_s[t % _NS],
                         add=True)

    def _wait_scat(t):
        pltpu.make_async_copy(ones_rows, deg_sh.at[_slice_d(t)],
                              sem_s[t % _NS]).wait()

    def _block(base, it, first):
        for t in range(_BLK):
            if not first or t >= _NS:
                _wait_scat((t - _NS) % _BLK)
            if t == _NS:
                _load_grp(1, base + _G * _CH)
            if t == _G:
                _wait_grp(1, base + _G * _CH)
            if t == _G + _NS:
                _load_grp(0, e_base + ((it + 1) % _NIT) * _BLK * _CH)
            if t == 0:
                _wait_grp(0, base)
            _scat(t)

    _load_grp(0, e_base)
    _block(e_base, 0, True)

    def body(it, carry):
        _block(e_base + it * _BLK * _CH, it, False)
        return carry

    lax.fori_loop(1, _NIT, body, 0)
    _wait_grp(0, e_base)
    for t in range(_BLK - _NS, _BLK):
        _wait_scat(t)
    plsc.subcore_barrier()
    pltpu.sync_copy(deg_sh.at[pl.ds(r0a, _RWIN)],
                    out_hbm.at[c, pl.ds(r0a, _RWIN)])


@functools.partial(
    pl.kernel,
    out_type=jax.ShapeDtypeStruct((2, _N, _F), jnp.float32),
    mesh=_mesh,
    compiler_params=_sc_params,
    scratch_types=(
        [pltpu.VMEM((_GB,), jnp.int32) for _ in range(4)]
        + [pltpu.VMEM((_CH, _F), jnp.float32) for _ in range(_NS)]
        + [pltpu.SemaphoreType.DMA for _ in range(2 * _NS + 2)]
        + [pltpu.VMEM_SHARED((_N, _F), jnp.float32)]
        + [pltpu.VMEM_SHARED((_N, _F), jnp.float32)]
    ),
)
def _sc_agg(y_hbm, ei_hbm, zeros_hbm, out_hbm, *refs):
    idx_s_ab = refs[0:2]
    idx_d_ab = refs[2:4]
    rows = refs[4:4 + _NS]
    sem_g = refs[4 + _NS:4 + 2 * _NS]
    sem_s = refs[4 + 2 * _NS:4 + 3 * _NS]
    sem_i = refs[4 + 3 * _NS:6 + 3 * _NS]
    y_sh = refs[6 + 3 * _NS]
    z_sh = refs[7 + 3 * _NS]
    c = lax.axis_index("c")
    s = lax.axis_index("s")
    wid = c * 16 + s
    r0a = _row_window(s)
    pltpu.sync_copy(y_hbm.at[pl.ds(r0a, _RWIN)], y_sh.at[pl.ds(r0a, _RWIN)])
    pltpu.sync_copy(zeros_hbm, z_sh.at[pl.ds(r0a, _RWIN)])
    plsc.subcore_barrier()
    e_base = wid * _EPW

    def _load_grp(p, off):
        pltpu.async_copy(ei_hbm.at[0, pl.ds(off, _GB)], idx_s_ab[p], sem_i[p])
        pltpu.async_copy(ei_hbm.at[1, pl.ds(off, _GB)], idx_d_ab[p], sem_i[p])

    def _wait_grp(p, off):
        pltpu.make_async_copy(ei_hbm.at[0, pl.ds(off, _GB)], idx_s_ab[p],
                              sem_i[p]).wait()
        pltpu.make_async_copy(ei_hbm.at[1, pl.ds(off, _GB)], idx_d_ab[p],
                              sem_i[p]).wait()

    def _slice_s(t):
        return idx_s_ab[t // _G].at[pl.ds((t % _G) * _CH, _CH)]

    def _slice_d(t):
        return idx_d_ab[t // _G].at[pl.ds((t % _G) * _CH, _CH)]

    def _gather(t):
        pltpu.async_copy(y_sh.at[_slice_s(t)], rows[t % _NS], sem_g[t % _NS])

    def _wait_gather(t):
        pltpu.make_async_copy(y_sh.at[_slice_s(t)], rows[t % _NS],
                              sem_g[t % _NS]).wait()

    def _scatter(t):
        pltpu.async_copy(rows[t % _NS], z_sh.at[_slice_d(t)], sem_s[t % _NS],
                         add=True)

    def _wait_scatter(t):
        pltpu.make_async_copy(rows[t % _NS], z_sh.at[_slice_d(t)],
                              sem_s[t % _NS]).wait()

    def _block(base, it, first):
        for t in range(_BLK):
            if not first or t >= _NS:
                _wait_scatter((t - _NS) % _BLK)
            if t == _NS:
                _load_grp(1, base + _G * _CH)
            if t == _G:
                _wait_grp(1, base + _G * _CH)
            if t == _G + _NS:
                _load_grp(0, e_base + ((it + 1) % _NIT) * _BLK * _CH)
            if t == 0:
                _wait_grp(0, base)
            _gather(t)
            if not first or t >= 2:
                _wait_gather((t - 2) % _BLK)
                _scatter((t - 2) % _BLK)

    _load_grp(0, e_base)
    _block(e_base, 0, True)

    def body(it, carry):
        _block(e_base + it * _BLK * _CH, it, False)
        return carry

    lax.fori_loop(1, _NIT, body, 0)
    _wait_grp(0, e_base)
    for t in (_BLK - 2, _BLK - 1):
        _wait_gather(t)
        _scatter(t)
    for t in range(_BLK - _NS, _BLK):
        _wait_scatter(t)
    plsc.subcore_barrier()
    pltpu.sync_copy(z_sh.at[pl.ds(r0a, _RWIN)],
                    out_hbm.at[c, pl.ds(r0a, _RWIN)])


def _t1_body(degp_ref, xpad_ref, sbc_ref, dinv_ref, xd_ref):
    deg = degp_ref[0] + degp_ref[1] + 1.0
    dinv16 = lax.rsqrt(deg)
    dinv = jnp.dot(dinv16, sbc_ref[...], preferred_element_type=jnp.float32)
    dinv_ref[...] = dinv
    xd_ref[...] = xpad_ref[...] * dinv


_t1 = pl.pallas_call(
    _t1_body,
    out_shape=[
        jax.ShapeDtypeStruct((_PR, 128), jnp.float32),
        jax.ShapeDtypeStruct((_PR, 128), jnp.float32),
    ],
)


def _t2_body(zp_ref, xd_ref, dinv_ref, w1_ref, b1_ref, w2_ref, y2_ref):
    w = (zp_ref[0] + zp_ref[1] + xd_ref[...]) * dinv_ref[...]
    h1 = jnp.dot(w, w1_ref[...], preferred_element_type=jnp.float32)
    h1 = jnp.maximum(h1 + b1_ref[...], 0.0)
    y2 = jnp.dot(h1, w2_ref[...], preferred_element_type=jnp.float32)
    y2_ref[...] = y2 * dinv_ref[...]


_t2 = pl.pallas_call(
    _t2_body,
    out_shape=jax.ShapeDtypeStruct((_PR, 128), jnp.float32),
)


def _t3_body(zp_ref, y2_ref, dinv_ref, b2_ref, wfc_ref, bfc_ref, out_ref):
    h2 = (zp_ref[0] + zp_ref[1] + y2_ref[...]) * dinv_ref[...]
    h2 = jnp.maximum(h2 + b2_ref[...], 0.0)
    o = jnp.dot(h2, wfc_ref[...], preferred_element_type=jnp.float32)
    out_ref[...] = jax.nn.sigmoid(o + bfc_ref[...])


_t3 = pl.pallas_call(
    _t3_body,
    out_shape=jax.ShapeDtypeStruct((_PR, 16), jnp.float32),
)


def kernel(x, edge_index, W1, b1, W2, b2, Wfc, bfc):
    f32 = jnp.float32
    ei = edge_index.astype(jnp.int32)
    zeros8 = jnp.zeros((_RWIN, _F), f32)
    ones1 = jnp.ones((_CH,), f32)
    zeros1 = jnp.zeros((_RWIN,), f32)

    degp = _sc_deg(ei, ones1, zeros1)
    xpad = jnp.concatenate([x.astype(f32), jnp.zeros((_N, 2), f32)], axis=1)
    sbc = jnp.kron(jnp.eye(16, dtype=f32), jnp.ones((1, 8), f32))
    dinv_p, xd_p = _t1(degp.reshape(2, _PR, 16), xpad.reshape(_PR, 128), sbc)

    z1p = _sc_agg(xd_p.reshape(_N, _F), ei, zeros8)

    eye16 = jnp.eye(16, dtype=f32)
    w1pad = jnp.concatenate([W1.astype(f32), jnp.zeros((2, 16), f32)], axis=0)
    w1big = jnp.kron(eye16, w1pad)
    b1big = jnp.tile(b1.astype(f32), 16).reshape(1, 256)
    w2big = jnp.kron(eye16, W2.astype(f32))
    y2_p = _t2(z1p.reshape(2, _PR, 128), xd_p, dinv_p, w1big, b1big, w2big)

    z2p = _sc_agg(y2_p.reshape(_N, _F), ei, zeros8)

    b2big = jnp.tile(b2.astype(f32), 16).reshape(1, 128)
    wfcbig = jnp.kron(eye16, Wfc.astype(f32))
    bfcbig = jnp.tile(bfc.astype(f32), 16).reshape(1, 16)
    out_p = _t3(z2p.reshape(2, _PR, 128), y2_p, dinv_p, b2big, wfcbig, bfcbig)
    return out_p.reshape(_N, 1)
